# baseline (speedup 1.0000x reference)
import functools

import jax
import jax.numpy as jnp
from jax import lax
from jax.experimental import pallas as pl
from jax.experimental.pallas import tpu as pltpu


def _key_to_f32(k):
    m = k ^ ((k >> 31) & jnp.int32(0x7FFFFFFF))
    return lax.bitcast_convert_type(m, jnp.float32)


def _mid(lo, hi):
    return (lo >> 1) + (hi >> 1) + (lo & hi & jnp.int32(1))


def _fused_body(x_ref, a_ref, t_ref, o_ref, th_ref, tm_ref, *, r_st, r_en,
                n_iter, n_sub, sub, c):
    j = pl.program_id(1)

    @pl.when(j == 0)
    def _bisect():
        def count_le2(ta, tb):
            ta = ta.reshape(1, c)
            tb = tb.reshape(1, c)

            def cbody(i, accs):
                acc_a, acc_b = accs
                xs = x_ref[0, pl.ds(i * sub, sub), :, :]
                w = xs.shape[1]
                xf = xs.reshape(sub * w, c)
                ma = (xf <= ta).astype(jnp.float32)
                mb = (xf <= tb).astype(jnp.float32)
                one = jnp.ones((1, sub * w), jnp.float32)
                acc_a = acc_a + jnp.dot(one, ma,
                                        preferred_element_type=jnp.float32)
                acc_b = acc_b + jnp.dot(one, mb,
                                        preferred_element_type=jnp.float32)
                return acc_a, acc_b

            z = jnp.zeros((1, c), jnp.float32)
            return lax.fori_loop(0, n_sub, cbody, (z, z))

        imin = jnp.full((1, c), jnp.iinfo(jnp.int32).min, jnp.int32)
        imax = jnp.full((1, c), jnp.iinfo(jnp.int32).max, jnp.int32)
        tgt1 = jnp.float32(r_st + 1)
        tgt2 = jnp.float32(r_en + 1)

        def step(_, state):
            lo1, hi1, lo2, hi2 = state
            m1 = _mid(lo1, hi1)
            m2 = _mid(lo2, hi2)
            c1, c2 = count_le2(_key_to_f32(m1), _key_to_f32(m2))
            p1 = c1 >= tgt1
            p2 = c2 >= tgt2
            lo1 = jnp.where(p1, lo1, m1 + 1)
            hi1 = jnp.where(p1, m1, hi1)
            lo2 = jnp.where(p2, lo2, m2 + 1)
            hi2 = jnp.where(p2, m2, hi2)
            return lo1, hi1, lo2, hi2

        lo1, _, lo2, _ = lax.fori_loop(0, n_iter, step,
                                       (imin, imax, imin, imax))
        st = _key_to_f32(lo1)
        en = _key_to_f32(lo2)

        th0 = st + (en - st) * a_ref[0]
        val0 = (th0 > 1e-14).astype(jnp.float32)
        th = th0 * val0
        val_st = th + (1.0 - val0)
        th_ref[...] = th
        tm_ref[...] = t_ref[0] / val_st

    xb = x_ref[0, pl.ds(j * sub, sub), :, :]
    th = th_ref[...].reshape(1, 1, c)
    tau_m = tm_ref[...].reshape(1, 1, c)
    o_ref[0] = jnp.maximum(xb, 0.0) / (
        1.0 + jnp.exp(-tau_m * (jnp.abs(xb) - th)))


def kernel(x, alpha, tau):
    B, H, W, C = x.shape
    L = H * W
    r_st = L - 1 - int(0.99 * L)
    r_en = L - 1 - int(0.01 * L)

    n_sub = 1
    for cand in (8, 7, 4, 2):
        if H % cand == 0 and H // cand >= 8:
            n_sub = cand
            break
    sub = H // n_sub

    body = functools.partial(_fused_body, r_st=r_st, r_en=r_en, n_iter=20,
                             n_sub=n_sub, sub=sub, c=C)
    y = pl.pallas_call(
        body,
        grid=(B, n_sub),
        in_specs=[
            pl.BlockSpec((1, H, W, C), lambda b, j: (b, 0, 0, 0)),
            pl.BlockSpec(memory_space=pltpu.SMEM),
            pl.BlockSpec(memory_space=pltpu.SMEM),
        ],
        out_specs=pl.BlockSpec((1, sub, W, C), lambda b, j: (b, j, 0, 0)),
        out_shape=jax.ShapeDtypeStruct((B, H, W, C), jnp.float32),
        scratch_shapes=[
            pltpu.VMEM((1, C), jnp.float32),
            pltpu.VMEM((1, C), jnp.float32),
        ],
    )(x, alpha, tau)
    return y


# speedup vs baseline: 1.0685x; 1.0685x over previous
import functools

import jax
import jax.numpy as jnp
from jax import lax
from jax.experimental import pallas as pl
from jax.experimental.pallas import tpu as pltpu


def _key_to_f32(k):
    m = k ^ ((k >> 31) & jnp.int32(0x7FFFFFFF))
    return lax.bitcast_convert_type(m, jnp.float32)


def _mid(lo, hi):
    return (lo >> 1) + (hi >> 1) + (lo & hi & jnp.int32(1))


def _thresh_body(x_ref, a_ref, t_ref, th_ref, tm_ref, *, r_st, r_en, n_iter,
                 n_sub, sub, c):

    def count_le2(ta, tb):
        ta = ta.reshape(1, c)
        tb = tb.reshape(1, c)

        def cbody(j, accs):
            acc_a, acc_b = accs
            xs = x_ref[0, pl.ds(j * sub, sub), :, :]
            w = xs.shape[1]
            xf = xs.reshape(sub * w, c)
            ma = (xf <= ta).astype(jnp.float32)
            mb = (xf <= tb).astype(jnp.float32)
            one = jnp.ones((1, sub * w), jnp.float32)
            acc_a = acc_a + jnp.dot(one, ma,
                                    preferred_element_type=jnp.float32)
            acc_b = acc_b + jnp.dot(one, mb,
                                    preferred_element_type=jnp.float32)
            return acc_a, acc_b

        z = jnp.zeros((1, c), jnp.float32)
        ca, cb = lax.fori_loop(0, n_sub, cbody, (z, z))
        return ca.reshape(1, 1, c), cb.reshape(1, 1, c)

    imin = jnp.full((1, 1, c), jnp.iinfo(jnp.int32).min, jnp.int32)
    imax = jnp.full((1, 1, c), jnp.iinfo(jnp.int32).max, jnp.int32)
    tgt1 = jnp.float32(r_st + 1)
    tgt2 = jnp.float32(r_en + 1)

    def step(_, state):
        lo1, hi1, lo2, hi2 = state
        m1 = _mid(lo1, hi1)
        m2 = _mid(lo2, hi2)
        c1, c2 = count_le2(_key_to_f32(m1), _key_to_f32(m2))
        p1 = c1 >= tgt1
        p2 = c2 >= tgt2
        lo1 = jnp.where(p1, lo1, m1 + 1)
        hi1 = jnp.where(p1, m1, hi1)
        lo2 = jnp.where(p2, lo2, m2 + 1)
        hi2 = jnp.where(p2, m2, hi2)
        return lo1, hi1, lo2, hi2

    lo1, _, lo2, _ = lax.fori_loop(0, n_iter, step, (imin, imax, imin, imax))
    st = _key_to_f32(lo1)
    en = _key_to_f32(lo2)

    th0 = st + (en - st) * a_ref[0]
    val0 = (th0 > 1e-14).astype(jnp.float32)
    th = th0 * val0
    val_st = th + (1.0 - val0)
    tau_m = t_ref[0] / val_st
    th_ref[0] = th
    tm_ref[0] = tau_m


def _prox_body(x_ref, th_ref, tm_ref, o_ref):
    xb = x_ref[0]
    th = th_ref[0]
    tau_m = tm_ref[0]
    o_ref[0] = jnp.maximum(xb, 0.0) / (
        1.0 + jnp.exp(-tau_m * (jnp.abs(xb) - th)))


def kernel(x, alpha, tau):
    B, H, W, C = x.shape
    L = H * W
    r_st = L - 1 - int(0.99 * L)
    r_en = L - 1 - int(0.01 * L)

    n_sub = 1
    for cand in (8, 7, 4, 2):
        if H % cand == 0 and H // cand >= 8:
            n_sub = cand
            break
    sub = H // n_sub

    tbody = functools.partial(_thresh_body, r_st=r_st, r_en=r_en, n_iter=18,
                              n_sub=n_sub, sub=sub, c=C)
    th, tm = pl.pallas_call(
        tbody,
        grid=(B,),
        in_specs=[
            pl.BlockSpec((1, H, W, C), lambda b: (b, 0, 0, 0)),
            pl.BlockSpec(memory_space=pltpu.SMEM),
            pl.BlockSpec(memory_space=pltpu.SMEM),
        ],
        out_specs=[
            pl.BlockSpec((1, 1, 1, C), lambda b: (b, 0, 0, 0)),
            pl.BlockSpec((1, 1, 1, C), lambda b: (b, 0, 0, 0)),
        ],
        out_shape=[
            jax.ShapeDtypeStruct((B, 1, 1, C), jnp.float32),
            jax.ShapeDtypeStruct((B, 1, 1, C), jnp.float32),
        ],
    )(x, alpha, tau)

    y = pl.pallas_call(
        _prox_body,
        grid=(B, n_sub),
        in_specs=[
            pl.BlockSpec((1, sub, W, C), lambda b, j: (b, j, 0, 0)),
            pl.BlockSpec((1, 1, 1, C), lambda b, j: (b, 0, 0, 0)),
            pl.BlockSpec((1, 1, 1, C), lambda b, j: (b, 0, 0, 0)),
        ],
        out_specs=pl.BlockSpec((1, sub, W, C), lambda b, j: (b, j, 0, 0)),
        out_shape=jax.ShapeDtypeStruct((B, H, W, C), jnp.float32),
    )(x, th, tm)
    return y
